# Initial kernel scaffold; baseline (speedup 1.0000x reference)
import jax
import jax.numpy as jnp
from jax.experimental import pallas as pl


def kernel(output, mask, ind, target):
    raise NotImplementedError("write your pallas kernel here")


# speedup vs baseline: 1.3869x; 1.3869x over previous
import functools

import jax
import jax.numpy as jnp
from jax import lax
from jax.experimental import pallas as pl
from jax.experimental.pallas import tpu as pltpu
from jax.experimental.pallas import tpu_sc as plsc

B, C, H, W, K = 32, 2, 128, 128, 128
HW = H * W
L = 16
ROWS_PER_MAP = HW // L


def _body(tab_hbm, mask_hbm, ind_hbm, tgt_hbm, psum_hbm, msum_hbm,
          idx_v, row0_v, row1_v, rem_v, g0_v, g1_v, t_v, m_v,
          res_v, mres_v, sem):
    b = lax.axis_index("s") * 2 + lax.axis_index("c")
    pltpu.sync_copy(ind_hbm.at[b], idx_v)
    iot = lax.iota(jnp.int32, L)
    base0 = (2 * b) * ROWS_PER_MAP
    base1 = base0 + ROWS_PER_MAP
    for j in range(K // L):
        sl = pl.ds(j * L, L)
        iv = idx_v[sl]
        r = lax.shift_right_logical(iv, 4)
        row0_v[sl] = r + base0
        row1_v[sl] = r + base1
        rem_v[sl] = lax.bitwise_and(iv, 15)
    cp0 = pltpu.async_copy(tab_hbm.at[row0_v], g0_v, sem)
    cp1 = pltpu.async_copy(tab_hbm.at[row1_v], g1_v, sem)
    pltpu.sync_copy(tgt_hbm.at[b], t_v)
    pltpu.sync_copy(mask_hbm.at[b], m_v)
    cp0.wait()
    cp1.wait()
    acc = jnp.zeros((L,), jnp.float32)
    macc = jnp.zeros((L,), jnp.float32)
    for j in range(K // L):
        sl = pl.ds(j * L, L)
        m = m_v[sl]
        rem = rem_v[sl]
        kk = iot + j * L
        p0 = plsc.load_gather(g0_v, [kk, rem])
        p1 = plsc.load_gather(g1_v, [kk, rem])
        i0 = iot * 2 + (j * 2 * L)
        t0 = plsc.load_gather(t_v, [i0])
        t1 = plsc.load_gather(t_v, [i0 + 1])
        acc = (acc
               + jnp.abs(p0 / (t0 + 1e-4) * m - m)
               + jnp.abs(p1 / (t1 + 1e-4) * m - m))
        macc = macc + m + m
    res_v[...] = acc
    mres_v[...] = macc
    pltpu.sync_copy(res_v, psum_hbm.at[b])
    pltpu.sync_copy(mres_v, msum_hbm.at[b])


@jax.jit
def kernel(output, mask, ind, target):
    tab = output.reshape(B * C * ROWS_PER_MAP, L)
    ind32 = ind.astype(jnp.int32)
    tflat = target.reshape(B, K * C)
    mesh = plsc.VectorSubcoreMesh(core_axis_name="c", subcore_axis_name="s")
    run = functools.partial(
        pl.kernel,
        mesh=mesh,
        compiler_params=pltpu.CompilerParams(
            needs_layout_passes=False, use_tc_tiling_on_sc=False),
        out_type=[
            jax.ShapeDtypeStruct((B, L), jnp.float32),
            jax.ShapeDtypeStruct((B, L), jnp.float32),
        ],
        scratch_types=[
            pltpu.VMEM((K,), jnp.int32),
            pltpu.VMEM((K,), jnp.int32),
            pltpu.VMEM((K,), jnp.int32),
            pltpu.VMEM((K,), jnp.int32),
            pltpu.VMEM((K, L), jnp.float32),
            pltpu.VMEM((K, L), jnp.float32),
            pltpu.VMEM((K * C,), jnp.float32),
            pltpu.VMEM((K,), jnp.float32),
            pltpu.VMEM((L,), jnp.float32),
            pltpu.VMEM((L,), jnp.float32),
            pltpu.SemaphoreType.DMA,
        ],
    )(_body)
    psum, msum = run(tab, mask, ind32, tflat)
    return psum.sum() / (msum.sum() + 1e-4)


# speedup vs baseline: 1.6709x; 1.2047x over previous
import functools

import jax
import jax.numpy as jnp
from jax import lax
from jax.experimental import pallas as pl
from jax.experimental.pallas import tpu as pltpu
from jax.experimental.pallas import tpu_sc as plsc

B, C, H, W, K = 32, 2, 128, 128, 128
HW = H * W
L = 16
ROWS_PER_MAP = HW // L
NS = 16


def _body(tab_hbm, mask_hbm, ind_hbm, tgt_hbm, out_hbm,
          idx_v, row0_v, row1_v, rem_v, g0_v, g1_v, t_v, m_v,
          acc_v, macc_v, all_v, loss_v, shared, sem):
    s = lax.axis_index("s")
    iot = lax.iota(jnp.int32, L)
    acc = jnp.zeros((L,), jnp.float32)
    macc = jnp.zeros((L,), jnp.float32)
    for i in range(B // NS):
        b = s * (B // NS) + i
        pltpu.sync_copy(ind_hbm.at[b], idx_v)
        base0 = (2 * b) * ROWS_PER_MAP
        base1 = base0 + ROWS_PER_MAP
        for j in range(K // L):
            sl = pl.ds(j * L, L)
            iv = idx_v[sl]
            r = lax.shift_right_logical(iv, 4)
            row0_v[sl] = r + base0
            row1_v[sl] = r + base1
            rem_v[sl] = lax.bitwise_and(iv, 15)
        cp0 = pltpu.async_copy(tab_hbm.at[row0_v], g0_v, sem)
        cp1 = pltpu.async_copy(tab_hbm.at[row1_v], g1_v, sem)
        pltpu.sync_copy(tgt_hbm.at[b], t_v)
        pltpu.sync_copy(mask_hbm.at[b], m_v)
        cp0.wait()
        cp1.wait()
        for j in range(K // L):
            sl = pl.ds(j * L, L)
            m = m_v[sl]
            rem = rem_v[sl]
            kk = iot + j * L
            p0 = plsc.load_gather(g0_v, [kk, rem])
            p1 = plsc.load_gather(g1_v, [kk, rem])
            i0 = iot * 2 + (j * 2 * L)
            t0 = plsc.load_gather(t_v, [i0])
            t1 = plsc.load_gather(t_v, [i0 + 1])
            acc = (acc
                   + jnp.abs(p0 / (t0 + 1e-4) * m - m)
                   + jnp.abs(p1 / (t1 + 1e-4) * m - m))
            macc = macc + m + m
    acc_v[...] = acc
    macc_v[...] = macc
    pltpu.sync_copy(acc_v, shared.at[s])
    pltpu.sync_copy(macc_v, shared.at[s + NS])
    plsc.subcore_barrier()

    @pl.when(s == 0)
    def _reduce():
        pltpu.sync_copy(shared, all_v)
        a = jnp.zeros((L,), jnp.float32)
        mm = jnp.zeros((L,), jnp.float32)
        for j in range(NS):
            a = a + all_v[j]
            mm = mm + all_v[j + NS]
        total = jnp.full((L,), jnp.sum(a))
        mtotal = jnp.full((L,), jnp.sum(mm))
        loss_v[...] = total / (mtotal + 1e-4)
        pltpu.sync_copy(loss_v.at[pl.ds(0, 8)], out_hbm)


@jax.jit
def kernel(output, mask, ind, target):
    tab = output.reshape(B * C * ROWS_PER_MAP, L)
    ind32 = ind.astype(jnp.int32)
    tflat = target.reshape(B, K * C)
    mesh = plsc.VectorSubcoreMesh(
        core_axis_name="c", subcore_axis_name="s", num_cores=1)
    run = functools.partial(
        pl.kernel,
        mesh=mesh,
        compiler_params=pltpu.CompilerParams(
            needs_layout_passes=False, use_tc_tiling_on_sc=False),
        out_type=jax.ShapeDtypeStruct((8,), jnp.float32),
        scratch_types=[
            pltpu.VMEM((K,), jnp.int32),
            pltpu.VMEM((K,), jnp.int32),
            pltpu.VMEM((K,), jnp.int32),
            pltpu.VMEM((K,), jnp.int32),
            pltpu.VMEM((K, L), jnp.float32),
            pltpu.VMEM((K, L), jnp.float32),
            pltpu.VMEM((K * C,), jnp.float32),
            pltpu.VMEM((K,), jnp.float32),
            pltpu.VMEM((L,), jnp.float32),
            pltpu.VMEM((L,), jnp.float32),
            pltpu.VMEM((2 * NS, L), jnp.float32),
            pltpu.VMEM((L,), jnp.float32),
            pltpu.VMEM_SHARED((2 * NS, L), jnp.float32),
            pltpu.SemaphoreType.DMA,
        ],
    )(_body)
    return run(tab, mask, ind32, tflat)[0]


# speedup vs baseline: 1.7809x; 1.0658x over previous
import functools

import jax
import jax.numpy as jnp
from jax import lax
from jax.experimental import pallas as pl
from jax.experimental.pallas import tpu as pltpu
from jax.experimental.pallas import tpu_sc as plsc

B, C, H, W, K = 32, 2, 128, 128, 128
HW = H * W
L = 16
ROWS_PER_MAP = HW // L
NS = 16


def _body(tab_hbm, mask_hbm, ind_hbm, tgt_hbm, out_hbm,
          idx_v, row0_v, row1_v, rem_v, g0_v, g1_v, t_v, m_v,
          acc_v, macc_v, all_v, loss_v, shared,
          sem_ind, sem_tm, sem_g0, sem_g1):
    s = lax.axis_index("s")
    iot = lax.iota(jnp.int32, L)
    nb = B // NS

    cp_ind = [pltpu.async_copy(ind_hbm.at[s * nb + i], idx_v.at[i], sem_ind)
              for i in range(nb)]
    cp_tm = [pltpu.async_copy(tgt_hbm.at[s * nb + i], t_v.at[i], sem_tm)
             for i in range(nb)]
    cp_tm += [pltpu.async_copy(mask_hbm.at[s * nb + i], m_v.at[i], sem_tm)
              for i in range(nb)]
    cp_g = []
    for i in range(nb):
        b = s * nb + i
        cp_ind[i].wait()
        base0 = (2 * b) * ROWS_PER_MAP
        base1 = base0 + ROWS_PER_MAP
        for j in range(K // L):
            sl = pl.ds(j * L, L)
            iv = idx_v[i, sl]
            r = lax.shift_right_logical(iv, 4)
            row0_v[i, sl] = r + base0
            row1_v[i, sl] = r + base1
            rem_v[i, sl] = lax.bitwise_and(iv, 15)
        sem_g = sem_g0 if i == 0 else sem_g1
        cp_g.append((pltpu.async_copy(tab_hbm.at[row0_v.at[i]], g0_v.at[i], sem_g),
                     pltpu.async_copy(tab_hbm.at[row1_v.at[i]], g1_v.at[i], sem_g)))
    for cp in cp_tm:
        cp.wait()
    acc = jnp.zeros((L,), jnp.float32)
    macc = jnp.zeros((L,), jnp.float32)
    for i in range(nb):
        cp_g[i][0].wait()
        cp_g[i][1].wait()
        for j in range(K // L):
            sl = pl.ds(j * L, L)
            m = m_v[i, sl]
            rem = rem_v[i, sl]
            kk = iot + j * L
            p0 = plsc.load_gather(g0_v, [jnp.full((L,), i), kk, rem])
            p1 = plsc.load_gather(g1_v, [jnp.full((L,), i), kk, rem])
            i0 = iot * 2 + (j * 2 * L)
            t0 = plsc.load_gather(t_v, [jnp.full((L,), i), i0])
            t1 = plsc.load_gather(t_v, [jnp.full((L,), i), i0 + 1])
            acc = (acc
                   + jnp.abs(p0 / (t0 + 1e-4) * m - m)
                   + jnp.abs(p1 / (t1 + 1e-4) * m - m))
            macc = macc + m + m
    acc_v[...] = acc
    macc_v[...] = macc
    pltpu.sync_copy(acc_v, shared.at[s])
    pltpu.sync_copy(macc_v, shared.at[s + NS])
    plsc.subcore_barrier()

    @pl.when(s == 0)
    def _reduce():
        pltpu.sync_copy(shared, all_v)
        a = jnp.zeros((L,), jnp.float32)
        mm = jnp.zeros((L,), jnp.float32)
        for j in range(NS):
            a = a + all_v[j]
            mm = mm + all_v[j + NS]
        total = jnp.full((L,), jnp.sum(a))
        mtotal = jnp.full((L,), jnp.sum(mm))
        loss_v[...] = total / (mtotal + 1e-4)
        pltpu.sync_copy(loss_v.at[pl.ds(0, 8)], out_hbm)


@jax.jit
def kernel(output, mask, ind, target):
    tab = output.reshape(B * C * ROWS_PER_MAP, L)
    ind32 = ind.astype(jnp.int32)
    tflat = target.reshape(B, K * C)
    mesh = plsc.VectorSubcoreMesh(
        core_axis_name="c", subcore_axis_name="s", num_cores=1)
    run = functools.partial(
        pl.kernel,
        mesh=mesh,
        compiler_params=pltpu.CompilerParams(
            needs_layout_passes=False, use_tc_tiling_on_sc=False),
        out_type=jax.ShapeDtypeStruct((8,), jnp.float32),
        scratch_types=[
            pltpu.VMEM((B // NS, K), jnp.int32),
            pltpu.VMEM((B // NS, K), jnp.int32),
            pltpu.VMEM((B // NS, K), jnp.int32),
            pltpu.VMEM((B // NS, K), jnp.int32),
            pltpu.VMEM((B // NS, K, L), jnp.float32),
            pltpu.VMEM((B // NS, K, L), jnp.float32),
            pltpu.VMEM((B // NS, K * C), jnp.float32),
            pltpu.VMEM((B // NS, K), jnp.float32),
            pltpu.VMEM((L,), jnp.float32),
            pltpu.VMEM((L,), jnp.float32),
            pltpu.VMEM((2 * NS, L), jnp.float32),
            pltpu.VMEM((L,), jnp.float32),
            pltpu.VMEM_SHARED((2 * NS, L), jnp.float32),
            pltpu.SemaphoreType.DMA,
            pltpu.SemaphoreType.DMA,
            pltpu.SemaphoreType.DMA,
            pltpu.SemaphoreType.DMA,
        ],
    )(_body)
    return run(tab, mask, ind32, tflat)[0]


# speedup vs baseline: 1.7872x; 1.0036x over previous
import functools

import jax
import jax.numpy as jnp
from jax import lax
from jax.experimental import pallas as pl
from jax.experimental.pallas import tpu as pltpu
from jax.experimental.pallas import tpu_sc as plsc

B, C, H, W, K = 32, 2, 128, 128, 128
HW = H * W
L = 16
ROWS_PER_MAP = HW // L
NS = 16


def _body(tab_hbm, mask_hbm, ind_hbm, tgt_hbm, out_hbm,
          idx_v, row0_v, row1_v, rem_v, g0_v, g1_v, t_v, m_v,
          acc_v, macc_v, all_v, loss_v, shared,
          sem_ind, sem_tm, sem_g0, sem_g1):
    s = lax.axis_index("s")
    iot = lax.iota(jnp.int32, L)
    nb = B // NS

    cp_ind = [pltpu.async_copy(ind_hbm.at[s * nb + i], idx_v.at[i], sem_ind)
              for i in range(nb)]
    cp_tm = [pltpu.async_copy(tgt_hbm.at[s * nb + i], t_v.at[i], sem_tm)
             for i in range(nb)]
    cp_tm += [pltpu.async_copy(mask_hbm.at[s * nb + i], m_v.at[i], sem_tm)
              for i in range(nb)]
    cp_g = []
    for i in range(nb):
        b = s * nb + i
        cp_ind[i].wait()
        base0 = (2 * b) * ROWS_PER_MAP
        base1 = base0 + ROWS_PER_MAP
        for j in range(K // L):
            sl = pl.ds(j * L, L)
            iv = idx_v[i, sl]
            r = lax.shift_right_logical(iv, 4)
            row0_v[i, sl] = r + base0
            row1_v[i, sl] = r + base1
            rem_v[i, sl] = lax.bitwise_and(iv, 15)
        sem_g = sem_g0 if i == 0 else sem_g1
        cp_g.append((pltpu.async_copy(tab_hbm.at[row0_v.at[i]], g0_v.at[i], sem_g),
                     pltpu.async_copy(tab_hbm.at[row1_v.at[i]], g1_v.at[i], sem_g)))
    for cp in cp_tm:
        cp.wait()
    acc = jnp.zeros((L,), jnp.float32)
    macc = jnp.zeros((L,), jnp.float32)
    for i in range(nb):
        cp_g[i][0].wait()
        cp_g[i][1].wait()
        for j in range(K // L):
            sl = pl.ds(j * L, L)
            m = m_v[i, sl]
            rem = rem_v[i, sl]
            kk = iot + j * L
            p0 = plsc.load_gather(g0_v, [jnp.full((L,), i), kk, rem])
            p1 = plsc.load_gather(g1_v, [jnp.full((L,), i), kk, rem])
            t0 = t_v[i, 0, sl]
            t1 = t_v[i, 1, sl]
            acc = (acc
                   + jnp.abs(p0 / (t0 + 1e-4) * m - m)
                   + jnp.abs(p1 / (t1 + 1e-4) * m - m))
            macc = macc + m + m
    acc_v[...] = acc
    macc_v[...] = macc
    pltpu.sync_copy(acc_v, shared.at[s])
    pltpu.sync_copy(macc_v, shared.at[s + NS])
    plsc.subcore_barrier()

    @pl.when(s == 0)
    def _reduce():
        pltpu.sync_copy(shared, all_v)
        a = jnp.zeros((L,), jnp.float32)
        mm = jnp.zeros((L,), jnp.float32)
        for j in range(NS):
            a = a + all_v[j]
            mm = mm + all_v[j + NS]
        total = jnp.full((L,), jnp.sum(a))
        mtotal = jnp.full((L,), jnp.sum(mm))
        loss_v[...] = total / (mtotal + 1e-4)
        pltpu.sync_copy(loss_v.at[pl.ds(0, 8)], out_hbm)


@jax.jit
def kernel(output, mask, ind, target):
    tab = output.reshape(B * C * ROWS_PER_MAP, L)
    ind32 = ind.astype(jnp.int32)
    tflat = jnp.transpose(target, (0, 2, 1))
    mesh = plsc.VectorSubcoreMesh(
        core_axis_name="c", subcore_axis_name="s", num_cores=1)
    run = functools.partial(
        pl.kernel,
        mesh=mesh,
        compiler_params=pltpu.CompilerParams(
            needs_layout_passes=False, use_tc_tiling_on_sc=False),
        out_type=jax.ShapeDtypeStruct((8,), jnp.float32),
        scratch_types=[
            pltpu.VMEM((B // NS, K), jnp.int32),
            pltpu.VMEM((B // NS, K), jnp.int32),
            pltpu.VMEM((B // NS, K), jnp.int32),
            pltpu.VMEM((B // NS, K), jnp.int32),
            pltpu.VMEM((B // NS, K, L), jnp.float32),
            pltpu.VMEM((B // NS, K, L), jnp.float32),
            pltpu.VMEM((B // NS, C, K), jnp.float32),
            pltpu.VMEM((B // NS, K), jnp.float32),
            pltpu.VMEM((L,), jnp.float32),
            pltpu.VMEM((L,), jnp.float32),
            pltpu.VMEM((2 * NS, L), jnp.float32),
            pltpu.VMEM((L,), jnp.float32),
            pltpu.VMEM_SHARED((2 * NS, L), jnp.float32),
            pltpu.SemaphoreType.DMA,
            pltpu.SemaphoreType.DMA,
            pltpu.SemaphoreType.DMA,
            pltpu.SemaphoreType.DMA,
        ],
    )(_body)
    return run(tab, mask, ind32, tflat)[0]


# speedup vs baseline: 1.7891x; 1.0011x over previous
import functools

import jax
import jax.numpy as jnp
from jax import lax
from jax.experimental import pallas as pl
from jax.experimental.pallas import tpu as pltpu
from jax.experimental.pallas import tpu_sc as plsc

B, C, H, W, K = 32, 2, 128, 128, 128
HW = H * W
L = 16
ROWS_PER_MAP = HW // L
NS = 16
NB = B // NS


def _body(tab_hbm, mask_hbm, row0_hbm, row1_hbm, rem_hbm, tgt_hbm, out_hbm,
          row0_v, row1_v, rem_v, g0_v, g1_v, t_v, m_v,
          acc_v, macc_v, all_v, loss_v, shared,
          sem_idx, sem_tm, sem_g0, sem_g1):
    s = lax.axis_index("s")
    iot = lax.iota(jnp.int32, L)

    cp_idx = []
    for i in range(NB):
        b = s * NB + i
        cp_idx.append((pltpu.async_copy(row0_hbm.at[b], row0_v.at[i], sem_idx),
                       pltpu.async_copy(row1_hbm.at[b], row1_v.at[i], sem_idx),
                       pltpu.async_copy(rem_hbm.at[b], rem_v.at[i], sem_idx)))
    cp_tm = []
    for i in range(NB):
        b = s * NB + i
        cp_tm.append(pltpu.async_copy(tgt_hbm.at[b], t_v.at[i], sem_tm))
        cp_tm.append(pltpu.async_copy(mask_hbm.at[b], m_v.at[i], sem_tm))
    cp_g = []
    for i in range(NB):
        cp_idx[i][0].wait()
        cp_idx[i][1].wait()
        sem_g = sem_g0 if i == 0 else sem_g1
        cp_g.append((pltpu.async_copy(tab_hbm.at[row0_v.at[i]], g0_v.at[i], sem_g),
                     pltpu.async_copy(tab_hbm.at[row1_v.at[i]], g1_v.at[i], sem_g)))
        cp_idx[i][2].wait()
    for cp in cp_tm:
        cp.wait()
    acc = jnp.zeros((L,), jnp.float32)
    macc = jnp.zeros((L,), jnp.float32)
    for i in range(NB):
        cp_g[i][0].wait()
        cp_g[i][1].wait()
        for j in range(K // L):
            sl = pl.ds(j * L, L)
            m = m_v[i, sl]
            rem = rem_v[i, sl]
            kk = iot + j * L
            p0 = plsc.load_gather(g0_v, [jnp.full((L,), i), kk, rem])
            p1 = plsc.load_gather(g1_v, [jnp.full((L,), i), kk, rem])
            t0 = t_v[i, 0, sl]
            t1 = t_v[i, 1, sl]
            acc = (acc
                   + jnp.abs(p0 / (t0 + 1e-4) * m - m)
                   + jnp.abs(p1 / (t1 + 1e-4) * m - m))
            macc = macc + m + m
    acc_v[...] = acc
    macc_v[...] = macc
    pltpu.sync_copy(acc_v, shared.at[s])
    pltpu.sync_copy(macc_v, shared.at[s + NS])
    plsc.subcore_barrier()

    @pl.when(s == 0)
    def _reduce():
        pltpu.sync_copy(shared, all_v)
        a = jnp.zeros((L,), jnp.float32)
        mm = jnp.zeros((L,), jnp.float32)
        for j in range(NS):
            a = a + all_v[j]
            mm = mm + all_v[j + NS]
        total = jnp.full((L,), jnp.sum(a))
        mtotal = jnp.full((L,), jnp.sum(mm))
        loss_v[...] = total / (mtotal + 1e-4)
        pltpu.sync_copy(loss_v.at[pl.ds(0, 8)], out_hbm)


@jax.jit
def kernel(output, mask, ind, target):
    tab = output.reshape(B * C * ROWS_PER_MAP, L)
    ind32 = ind.astype(jnp.int32)
    plane = jnp.arange(B, dtype=jnp.int32)[:, None] * (C * ROWS_PER_MAP)
    row0 = (ind32 >> 4) + plane
    row1 = row0 + ROWS_PER_MAP
    rem = ind32 & 15
    tflat = jnp.transpose(target, (0, 2, 1))
    mesh = plsc.VectorSubcoreMesh(
        core_axis_name="c", subcore_axis_name="s", num_cores=1)
    run = functools.partial(
        pl.kernel,
        mesh=mesh,
        compiler_params=pltpu.CompilerParams(
            needs_layout_passes=False, use_tc_tiling_on_sc=False),
        out_type=jax.ShapeDtypeStruct((8,), jnp.float32),
        scratch_types=[
            pltpu.VMEM((NB, K), jnp.int32),
            pltpu.VMEM((NB, K), jnp.int32),
            pltpu.VMEM((NB, K), jnp.int32),
            pltpu.VMEM((NB, K, L), jnp.float32),
            pltpu.VMEM((NB, K, L), jnp.float32),
            pltpu.VMEM((NB, C, K), jnp.float32),
            pltpu.VMEM((NB, K), jnp.float32),
            pltpu.VMEM((L,), jnp.float32),
            pltpu.VMEM((L,), jnp.float32),
            pltpu.VMEM((2 * NS, L), jnp.float32),
            pltpu.VMEM((L,), jnp.float32),
            pltpu.VMEM_SHARED((2 * NS, L), jnp.float32),
            pltpu.SemaphoreType.DMA,
            pltpu.SemaphoreType.DMA,
            pltpu.SemaphoreType.DMA,
            pltpu.SemaphoreType.DMA,
        ],
    )(_body)
    return run(tab, mask, row0, row1, rem, tflat)[0]


# speedup vs baseline: 1.8058x; 1.0093x over previous
import functools

import jax
import jax.numpy as jnp
from jax import lax
from jax.experimental import pallas as pl
from jax.experimental.pallas import tpu as pltpu
from jax.experimental.pallas import tpu_sc as plsc

B, C, H, W, K = 32, 2, 128, 128, 128
HW = H * W
L = 16
ROWS_PER_MAP = HW // L
NS = 16
NB = B // NS


def _body(tab_hbm, mask_hbm, row0_hbm, row1_hbm, rem_hbm, tgt_hbm, out_hbm,
          row0_v, row1_v, rem_v, g0_v, g1_v, t_v, m_v,
          acc_v, macc_v, all_v, loss_v, shared,
          sem_idx, sem_tm, sem_g0, sem_g1):
    s = lax.axis_index("s")
    iot = lax.iota(jnp.int32, L)

    cp_idx = []
    for i in range(NB):
        b = s * NB + i
        cp_idx.append((pltpu.async_copy(row0_hbm.at[b], row0_v.at[i], sem_idx),
                       pltpu.async_copy(row1_hbm.at[b], row1_v.at[i], sem_idx),
                       pltpu.async_copy(rem_hbm.at[b], rem_v.at[i], sem_idx)))
    cp_tm = []
    for i in range(NB):
        b = s * NB + i
        cp_tm.append(pltpu.async_copy(tgt_hbm.at[b], t_v.at[i], sem_tm))
        cp_tm.append(pltpu.async_copy(mask_hbm.at[b], m_v.at[i], sem_tm))
    cp_g = []
    for i in range(NB):
        cp_idx[i][0].wait()
        cp_idx[i][1].wait()
        sem_g = sem_g0 if i == 0 else sem_g1
        cp_g.append((pltpu.async_copy(tab_hbm.at[row0_v.at[i]], g0_v.at[i], sem_g),
                     pltpu.async_copy(tab_hbm.at[row1_v.at[i]], g1_v.at[i], sem_g)))
        cp_idx[i][2].wait()
    for cp in cp_tm:
        cp.wait()
    acc = jnp.zeros((L,), jnp.float32)
    macc = jnp.zeros((L,), jnp.float32)
    for i in range(NB):
        cp_g[i][0].wait()
        cp_g[i][1].wait()

        def chunk(j, carry, i=i):
            a, ma = carry
            sl = pl.ds(j * L, L)
            m = m_v[i, sl]
            rem = rem_v[i, sl]
            kk = iot + j * L
            p0 = plsc.load_gather(g0_v, [jnp.full((L,), i), kk, rem])
            p1 = plsc.load_gather(g1_v, [jnp.full((L,), i), kk, rem])
            t0 = t_v[i, 0, sl]
            t1 = t_v[i, 1, sl]
            a = (a
                 + jnp.abs(p0 / (t0 + 1e-4) * m - m)
                 + jnp.abs(p1 / (t1 + 1e-4) * m - m))
            return a, ma + m + m

        acc, macc = lax.fori_loop(0, K // L, chunk, (acc, macc))
    acc_v[...] = acc
    macc_v[...] = macc
    pltpu.sync_copy(acc_v, shared.at[s])
    pltpu.sync_copy(macc_v, shared.at[s + NS])
    plsc.subcore_barrier()

    @pl.when(s == 0)
    def _reduce():
        pltpu.sync_copy(shared, all_v)

        def acc_row(j, carry):
            a, mm = carry
            return a + all_v[j], mm + all_v[j + NS]

        a, mm = lax.fori_loop(
            0, NS, acc_row,
            (jnp.zeros((L,), jnp.float32), jnp.zeros((L,), jnp.float32)))
        total = jnp.full((L,), jnp.sum(a))
        mtotal = jnp.full((L,), jnp.sum(mm))
        loss_v[...] = total / (mtotal + 1e-4)
        pltpu.sync_copy(loss_v.at[pl.ds(0, 8)], out_hbm)


@jax.jit
def kernel(output, mask, ind, target):
    tab = output.reshape(B * C * ROWS_PER_MAP, L)
    ind32 = ind.astype(jnp.int32)
    plane = jnp.arange(B, dtype=jnp.int32)[:, None] * (C * ROWS_PER_MAP)
    row0 = (ind32 >> 4) + plane
    row1 = row0 + ROWS_PER_MAP
    rem = ind32 & 15
    tflat = jnp.transpose(target, (0, 2, 1))
    mesh = plsc.VectorSubcoreMesh(
        core_axis_name="c", subcore_axis_name="s", num_cores=1)
    run = functools.partial(
        pl.kernel,
        mesh=mesh,
        compiler_params=pltpu.CompilerParams(
            needs_layout_passes=False, use_tc_tiling_on_sc=False),
        out_type=jax.ShapeDtypeStruct((8,), jnp.float32),
        scratch_types=[
            pltpu.VMEM((NB, K), jnp.int32),
            pltpu.VMEM((NB, K), jnp.int32),
            pltpu.VMEM((NB, K), jnp.int32),
            pltpu.VMEM((NB, K, L), jnp.float32),
            pltpu.VMEM((NB, K, L), jnp.float32),
            pltpu.VMEM((NB, C, K), jnp.float32),
            pltpu.VMEM((NB, K), jnp.float32),
            pltpu.VMEM((L,), jnp.float32),
            pltpu.VMEM((L,), jnp.float32),
            pltpu.VMEM((2 * NS, L), jnp.float32),
            pltpu.VMEM((L,), jnp.float32),
            pltpu.VMEM_SHARED((2 * NS, L), jnp.float32),
            pltpu.SemaphoreType.DMA,
            pltpu.SemaphoreType.DMA,
            pltpu.SemaphoreType.DMA,
            pltpu.SemaphoreType.DMA,
        ],
    )(_body)
    return run(tab, mask, row0, row1, rem, tflat)[0]
